# baseline (speedup 1.0000x reference)
import functools

import jax
import jax.numpy as jnp
from jax import lax
from jax.experimental import pallas as pl
from jax.experimental.pallas import tpu as pltpu
from jax.experimental.pallas import tpu_sc as plsc

LINPUT = 256
COUNT = 10
BUCKET_SIZE = 6
BATCH = 16384

NC = 2
NS = 16
NW = NC * NS
BPW = BATCH // NW
CHUNK = 128
NCHUNK = BPW // CHUNK
GROUPS = CHUNK // 16
MCH = LINPUT // 16


def _make_sc_kernel():
    mesh = plsc.VectorSubcoreMesh(core_axis_name="c", subcore_axis_name="s")

    @functools.partial(
        pl.kernel,
        mesh=mesh,
        out_type=jax.ShapeDtypeStruct((BATCH,), jnp.float32),
        compiler_params=pltpu.CompilerParams(needs_layout_passes=False),
        scratch_types=[
            pltpu.VMEM((COUNT * LINPUT + 16,), jnp.float32),
            pltpu.VMEM((BPW,), jnp.int32),
            pltpu.VMEM((BPW,), jnp.float32),
            pltpu.VMEM((CHUNK * LINPUT,), jnp.float32),
            pltpu.VMEM((CHUNK * LINPUT,), jnp.float32),
            pltpu.SemaphoreType.DMA,
            pltpu.SemaphoreType.DMA,
        ],
    )
    def k(x_hbm, ply_hbm, w_hbm, out_hbm,
          w_v, ply_v, out_v, xb0, xb1, sem0, sem1):
        wid = lax.axis_index("s") * NC + lax.axis_index("c")
        base = wid * BPW

        xbufs = (xb0, xb1)
        sems = (sem0, sem1)

        cps = [None, None]
        cps[0] = pltpu.async_copy(
            x_hbm.at[pl.ds(base * LINPUT, CHUNK * LINPUT)], xb0, sem0)
        pltpu.sync_copy(w_hbm, w_v)
        pltpu.sync_copy(ply_hbm.at[pl.ds(base, BPW)], ply_v)

        iota16 = lax.iota(jnp.int32, 16)
        lane_masks = [iota16 == j for j in range(16)]
        bias_v = w_v[pl.ds(COUNT * LINPUT, 16)]

        for c in range(NCHUNK):
            cur = c % 2
            cps[cur].wait()
            if c + 1 < NCHUNK:
                nxt = (c + 1) % 2
                cps[nxt] = pltpu.async_copy(
                    x_hbm.at[pl.ds((base + (c + 1) * CHUNK) * LINPUT,
                                   CHUNK * LINPUT)],
                    xbufs[nxt], sems[nxt])
            x_v = xbufs[cur]

            @plsc.parallel_loop(0, GROUPS, unroll=2)
            def gbody(g):
                plyv = ply_v[pl.ds(c * CHUNK + g * 16, 16)]
                idxv = lax.div(plyv, jnp.int32(BUCKET_SIZE))
                wbasev = idxv * LINPUT
                outvec = jnp.take_along_axis(
                    bias_v, idxv, axis=0, mode="promise_in_bounds")
                goff = g * (16 * LINPUT)
                for j in range(16):
                    wb = wbasev[j]
                    xoff = goff + j * LINPUT
                    acc = (x_v[pl.ds(xoff, 16)] * w_v[pl.ds(wb, 16)] +
                           x_v[pl.ds(xoff + 16, 16)] * w_v[pl.ds(wb + 16, 16)])
                    for m in range(2, MCH):
                        acc = acc + (x_v[pl.ds(xoff + m * 16, 16)] *
                                     w_v[pl.ds(wb + m * 16, 16)])
                    res = jnp.sum(acc)
                    outvec = jnp.where(lane_masks[j], res, outvec)
                out_v[pl.ds(c * CHUNK + g * 16, 16)] = outvec

        pltpu.sync_copy(out_v, out_hbm.at[pl.ds(base, BPW)])

    return k


_sc_kernel = _make_sc_kernel()


@jax.jit
def kernel(x_pa, ply, W, b):
    x_flat = x_pa.reshape(BATCH * LINPUT)
    wb_flat = jnp.concatenate(
        [W.reshape(COUNT * LINPUT),
         jnp.pad(b.reshape(COUNT), (0, 16 - COUNT))])
    out = _sc_kernel(x_flat, ply, wb_flat)
    return out.reshape(BATCH, 1)


# speedup vs baseline: 2.1095x; 2.1095x over previous
import functools

import jax
import jax.numpy as jnp
from jax import lax
from jax.experimental import pallas as pl
from jax.experimental.pallas import tpu as pltpu
from jax.experimental.pallas import tpu_sc as plsc

LINPUT = 256
COUNT = 10
BUCKET_SIZE = 6
BATCH = 16384
NSTACK = 16

NC = 2
NS = 16
NW = NC * NS
BPW = BATCH // NW
ROWS_PER_BLOCK = 1024


def _tc_matmul_body(x_ref, wt_ref, b_ref, o_ref):
    o_ref[...] = (
        jnp.dot(x_ref[...], wt_ref[...], preferred_element_type=jnp.float32)
        + b_ref[...])


_tc_matmul = pl.pallas_call(
    _tc_matmul_body,
    grid=(BATCH // ROWS_PER_BLOCK,),
    in_specs=[
        pl.BlockSpec((ROWS_PER_BLOCK, LINPUT), lambda i: (i, 0)),
        pl.BlockSpec((LINPUT, NSTACK), lambda i: (0, 0)),
        pl.BlockSpec((1, NSTACK), lambda i: (0, 0)),
    ],
    out_specs=pl.BlockSpec((ROWS_PER_BLOCK, NSTACK), lambda i: (i, 0)),
    out_shape=jax.ShapeDtypeStruct((BATCH, NSTACK), jnp.float32),
)


def _make_sc_select():
    mesh = plsc.VectorSubcoreMesh(core_axis_name="c", subcore_axis_name="s")

    @functools.partial(
        pl.kernel,
        mesh=mesh,
        out_type=jax.ShapeDtypeStruct((BATCH,), jnp.float32),
        compiler_params=pltpu.CompilerParams(needs_layout_passes=False),
        scratch_types=[
            pltpu.VMEM((BPW * NSTACK,), jnp.float32),
            pltpu.VMEM((BPW,), jnp.int32),
            pltpu.VMEM((BPW,), jnp.float32),
            pltpu.SemaphoreType.DMA,
        ],
    )
    def k(lg_hbm, ply_hbm, out_hbm, lg_v, ply_v, out_v, sem):
        wid = lax.axis_index("s") * NC + lax.axis_index("c")
        base = wid * BPW

        cp = pltpu.async_copy(
            lg_hbm.at[pl.ds(base * NSTACK, BPW * NSTACK)], lg_v, sem)
        pltpu.sync_copy(ply_hbm.at[pl.ds(base, BPW)], ply_v)
        cp.wait()

        iota16 = lax.iota(jnp.int32, 16)
        for g in range(BPW // 16):
            plyv = ply_v[pl.ds(g * 16, 16)]
            idxv = lax.div(plyv, jnp.int32(BUCKET_SIZE))
            addr = (g * 16 * NSTACK) + iota16 * NSTACK + idxv
            out_v[pl.ds(g * 16, 16)] = plsc.load_gather(lg_v, [addr])

        pltpu.sync_copy(out_v, out_hbm.at[pl.ds(base, BPW)])

    return k


_sc_select = _make_sc_select()


@jax.jit
def kernel(x_pa, ply, W, b):
    wt = jnp.zeros((LINPUT, NSTACK), jnp.float32)
    wt = wt.at[:, :COUNT].set(W.reshape(COUNT, LINPUT).T)
    bp = jnp.zeros((1, NSTACK), jnp.float32).at[0, :COUNT].set(b.reshape(COUNT))
    logits = _tc_matmul(x_pa, wt, bp)
    out = _sc_select(logits.reshape(BATCH * NSTACK), ply)
    return out.reshape(BATCH, 1)
